# baseline (speedup 1.0000x reference)
import functools

import jax
import jax.numpy as jnp
from jax import lax
from jax.experimental import pallas as pl
from jax.experimental.pallas import tpu as pltpu
from jax.experimental.pallas import tpu_sc as plsc

D = 64
CHUNK = 128
NBUF = 8


def _make_lookup(B, V):
    info = plsc.get_sparse_core_info()
    NC, NS = info.num_cores, info.num_subcores
    NW = NC * NS
    assert B % (NW * CHUNK) == 0
    nchunk = B // (NW * CHUNK)
    assert nchunk % NBUF == 0
    mesh = plsc.VectorSubcoreMesh(core_axis_name="c", subcore_axis_name="s")

    @functools.partial(
        pl.kernel,
        mesh=mesh,
        out_type=jax.ShapeDtypeStruct((NW * nchunk, CHUNK, D), jnp.float32),
        scratch_types=[
            pltpu.VMEM((nchunk, CHUNK), jnp.int32),
            pltpu.VMEM((NBUF, CHUNK, D), jnp.float32),
            pltpu.SemaphoreType.DMA,
            pltpu.SemaphoreType.DMA,
        ],
        compiler_params=pltpu.CompilerParams(use_tc_tiling_on_sc=False),
    )
    def lookup(table_hbm, idx_hbm, out_hbm, idx_v, rows_v, gsem, ssem):
        wid = lax.axis_index("s") * NC + lax.axis_index("c")
        pltpu.sync_copy(idx_hbm.at[wid], idx_v)

        def step(g, carry):
            base = g * NBUF
            for b in range(NBUF):
                pltpu.async_copy(
                    table_hbm.at[idx_v.at[base + b]], rows_v.at[b], gsem)
            for b in range(NBUF):
                pltpu.make_async_copy(
                    table_hbm.at[idx_v.at[base + b]], rows_v.at[b], gsem).wait()
            for b in range(NBUF):
                pltpu.async_copy(
                    rows_v.at[b], out_hbm.at[wid * nchunk + base + b], ssem)
            for b in range(NBUF):
                pltpu.make_async_copy(
                    rows_v.at[b], out_hbm.at[wid * nchunk + base + b], ssem).wait()
            return carry

        lax.fori_loop(0, nchunk // NBUF, step, 0)

    return lookup


def kernel(inputs, token_emb):
    S0, S1 = inputs.shape
    B = S0 * S1
    V = token_emb.shape[0]
    info = plsc.get_sparse_core_info()
    NW = info.num_cores * info.num_subcores
    nchunk = B // (NW * CHUNK)
    idx = inputs.reshape(NW, nchunk, CHUNK).astype(jnp.int32)
    out = _make_lookup(B, V)(token_emb, idx)
    return out.reshape(S0, S1, D)


# speedup vs baseline: 1.0066x; 1.0066x over previous
import functools

import jax
import jax.numpy as jnp
from jax import lax
from jax.experimental import pallas as pl
from jax.experimental.pallas import tpu as pltpu
from jax.experimental.pallas import tpu_sc as plsc

D = 64
CHUNK = 128
K = 4


def _make_lookup(B, V):
    info = plsc.get_sparse_core_info()
    NC, NS = info.num_cores, info.num_subcores
    NW = NC * NS
    assert B % (NW * CHUNK) == 0
    nchunk = B // (NW * CHUNK)
    G = nchunk // K
    assert nchunk % K == 0 and G % 2 == 0
    mesh = plsc.VectorSubcoreMesh(core_axis_name="c", subcore_axis_name="s")

    @functools.partial(
        pl.kernel,
        mesh=mesh,
        out_type=jax.ShapeDtypeStruct((NW * nchunk, CHUNK, D), jnp.float32),
        scratch_types=[
            pltpu.VMEM((nchunk, CHUNK), jnp.int32),
            pltpu.VMEM((2 * K, CHUNK, D), jnp.float32),
            pltpu.SemaphoreType.DMA,
            pltpu.SemaphoreType.DMA,
        ],
        compiler_params=pltpu.CompilerParams(use_tc_tiling_on_sc=False),
    )
    def lookup(table_hbm, idx_hbm, out_hbm, idx_v, rows_v, gsem, ssem):
        wid = lax.axis_index("s") * NC + lax.axis_index("c")
        obase = wid * nchunk
        pltpu.sync_copy(idx_hbm.at[wid], idx_v)

        def fire_gathers(g, H):
            for b in range(K):
                pltpu.async_copy(
                    table_hbm.at[idx_v.at[g * K + b]], rows_v.at[H * K + b], gsem)

        def drain_gathers(g, H):
            for b in range(K):
                pltpu.make_async_copy(
                    table_hbm.at[idx_v.at[g * K + b]], rows_v.at[H * K + b], gsem).wait()

        def fire_stores(g, H):
            for b in range(K):
                pltpu.async_copy(
                    rows_v.at[H * K + b], out_hbm.at[obase + g * K + b], ssem)

        def drain_stores(g, H):
            for b in range(K):
                pltpu.make_async_copy(
                    rows_v.at[H * K + b], out_hbm.at[obase + g * K + b], ssem).wait()

        fire_gathers(0, 0)
        drain_gathers(0, 0)
        fire_stores(0, 0)
        fire_gathers(1, 1)

        def group(g, H):
            drain_gathers(g, H)
            fire_stores(g, H)
            drain_stores(g - 1, 1 - H)
            fire_gathers(g + 1, 1 - H)

        def pair(t, carry):
            group(2 * t + 1, 1)
            group(2 * t + 2, 0)
            return carry

        lax.fori_loop(0, (G - 2) // 2, pair, 0)

        g = G - 1
        drain_gathers(g, 1)
        fire_stores(g, 1)
        drain_stores(g - 1, 0)
        drain_stores(g, 1)

    return lookup


def kernel(inputs, token_emb):
    S0, S1 = inputs.shape
    B = S0 * S1
    V = token_emb.shape[0]
    info = plsc.get_sparse_core_info()
    NW = info.num_cores * info.num_subcores
    nchunk = B // (NW * CHUNK)
    idx = inputs.reshape(NW, nchunk, CHUNK).astype(jnp.int32)
    out = _make_lookup(B, V)(token_emb, idx)
    return out.reshape(S0, S1, D)


# speedup vs baseline: 1.0076x; 1.0010x over previous
import functools

import jax
import jax.numpy as jnp
from jax import lax
from jax.experimental import pallas as pl
from jax.experimental.pallas import tpu as pltpu
from jax.experimental.pallas import tpu_sc as plsc

D = 64
CHUNK = 128
K = 4


def _make_lookup(B, V):
    info = plsc.get_sparse_core_info()
    NC, NS = info.num_cores, info.num_subcores
    NW = NC * NS
    assert B % (NW * CHUNK) == 0
    nchunk = B // (NW * CHUNK)
    G = nchunk // K
    assert nchunk % K == 0 and G % 2 == 0
    mesh = plsc.VectorSubcoreMesh(core_axis_name="c", subcore_axis_name="s")

    @functools.partial(
        pl.kernel,
        mesh=mesh,
        out_type=jax.ShapeDtypeStruct((NW * nchunk * CHUNK, D), jnp.float32),
        scratch_types=[
            pltpu.VMEM((nchunk, CHUNK), jnp.int32),
            pltpu.VMEM((2 * K, CHUNK, D), jnp.float32),
            pltpu.SemaphoreType.DMA,
            pltpu.SemaphoreType.DMA,
        ],
        compiler_params=pltpu.CompilerParams(use_tc_tiling_on_sc=False),
    )
    def lookup(table_hbm, idx_hbm, out_hbm, idx_v, rows_v, gsem, ssem):
        wid = lax.axis_index("s") * NC + lax.axis_index("c")
        obase = wid * nchunk
        pltpu.sync_copy(idx_hbm.at[wid], idx_v)

        def fire_gathers(g, H):
            for b in range(K):
                pltpu.async_copy(
                    table_hbm.at[idx_v.at[g * K + b]], rows_v.at[H * K + b], gsem)

        def drain_gathers(g, H):
            for b in range(K):
                pltpu.make_async_copy(
                    table_hbm.at[idx_v.at[g * K + b]], rows_v.at[H * K + b], gsem).wait()

        def fire_stores(g, H):
            for b in range(K):
                pltpu.async_copy(
                    rows_v.at[H * K + b],
                    out_hbm.at[pl.ds((obase + g * K + b) * CHUNK, CHUNK)], ssem)

        def drain_stores(g, H):
            for b in range(K):
                pltpu.make_async_copy(
                    rows_v.at[H * K + b],
                    out_hbm.at[pl.ds((obase + g * K + b) * CHUNK, CHUNK)], ssem).wait()

        fire_gathers(0, 0)
        drain_gathers(0, 0)
        fire_stores(0, 0)
        fire_gathers(1, 1)

        def group(g, H):
            drain_gathers(g, H)
            fire_stores(g, H)
            drain_stores(g - 1, 1 - H)
            fire_gathers(g + 1, 1 - H)

        def pair(t, carry):
            group(2 * t + 1, 1)
            group(2 * t + 2, 0)
            return carry

        lax.fori_loop(0, (G - 2) // 2, pair, 0)

        g = G - 1
        drain_gathers(g, 1)
        fire_stores(g, 1)
        drain_stores(g - 1, 0)
        drain_stores(g, 1)

    return lookup


def kernel(inputs, token_emb):
    S0, S1 = inputs.shape
    B = S0 * S1
    V = token_emb.shape[0]
    info = plsc.get_sparse_core_info()
    NW = info.num_cores * info.num_subcores
    nchunk = B // (NW * CHUNK)
    idx = inputs.reshape(NW, nchunk, CHUNK).astype(jnp.int32)
    out = _make_lookup(B, V)(token_emb, idx)
    return out.reshape(S0, S1, D)
